# baseline (speedup 1.0000x reference)
import functools

import jax
import jax.numpy as jnp
import numpy as np
from jax import lax
from jax.experimental import pallas as pl
from jax.experimental.pallas import tpu as pltpu
from jax.experimental.pallas import tpu_sc as plsc

N = 4096
NPARAMS = N * (N - 1) // 2
B = 128
D64 = 64
RPS = 3
SROWS = RPS * B
M = NPARAMS // D64
NB = N // B
NW = 32
LANES = 16

_pairs = [(bi, bj) for bi in range(NB) for bj in range(bi + 1, NB)]
NPAIR = len(_pairs)
TSLOTS = -(-NPAIR // NW)
PADP = NW * TSLOTS + LANES
_bi_tab = np.full((PADP,), 0, np.int32)
_bj_tab = np.full((PADP,), 1, np.int32)
for _p, (_a, _b) in enumerate(_pairs):
    _bi_tab[_p] = _a
    _bj_tab[_p] = _b


def _offset(i):
    return i * (N - 1) - ((i * (i - 1)) >> 1)


def _body(ptab_hbm, bi_hbm, bj_hbm, out_hbm, tabi, tabj,
          sblk0, sblk1, idx0, idx1, ublk0, ublk1, tblk0, tblk1, dbuf,
          sem_in0, sem_in1, sem_out0, sem_out1):
    cid = lax.axis_index("c")
    sid = lax.axis_index("s")
    wid = sid * 2 + cid

    SB = (sblk0, sblk1)
    IX = (idx0, idx1)
    UB = (ublk0, ublk1)
    TB = (tblk0, tblk1)
    SI = (sem_in0, sem_in1)
    SO = (sem_out0, sem_out1)

    pltpu.sync_copy(bi_hbm, tabi)
    pltpu.sync_copy(bj_hbm, tabj)

    iota = lax.iota(jnp.int32, LANES)

    def pair_of(p):
        bi = tabi[pl.ds(p, LANES)][0]
        bj = tabj[pl.ds(p, LANES)][0]
        return bi, bj

    def pair_start(bi, bj):
        def start_of_row(r):
            i = bi * B + r
            return _offset(i) + bj * B - i - 1

        return start_of_row

    def diag_start(r):
        return jnp.maximum(0, _offset(wid * B + r) - r - 1)

    def gbase_of(s):
        return jnp.minimum(s >> 6, M - RPS)

    def fire_block(start_of_row, par):
        idxb = IX[par]
        for k in range(B // LANES):
            rows = k * LANES + iota
            g = gbase_of(start_of_row(rows))
            base = rows * RPS
            for q in range(RPS):
                plsc.store_scatter(idxb, [base + q], g + q)
        for q in range(RPS):
            pltpu.make_async_copy(
                ptab_hbm.at[idxb.at[pl.ds(q * B, B)]],
                SB[par].at[pl.ds(q * B, B)],
                SI[par],
            ).start()

    def drain_in(par):
        pltpu.make_async_copy(
            out_hbm.at[pl.ds(0, SROWS), pl.ds(0, D64)], SB[par], SI[par]
        ).wait()

    def drain_out(par):
        pltpu.make_async_copy(
            out_hbm.at[pl.ds(0, B), pl.ds(0, B)], TB[par], SO[par]
        ).wait()

    def fire_pair(jt, par):
        p = wid + NW * jt

        @pl.when(p < NPAIR)
        def _():
            bi, bj = pair_of(p)
            fire_block(pair_start(bi, bj), par)

    def compute_pair(jt, tt, par):
        p = wid + NW * jt

        @pl.when(p < NPAIR)
        def _():
            drain_in(par)

            @pl.when(tt >= 1)
            def _():
                drain_out(par)
                drain_out(par)

            bi, bj = pair_of(p)
            r0 = pl.multiple_of(bi * B, B)
            c0 = pl.multiple_of(bj * B, B)
            start_of_row = pair_start(bi, bj)
            sblk, ublk, tblk = SB[par], UB[par], TB[par]

            def realign(r, carry):
                s = start_of_row(r)
                g = gbase_of(s)
                d = s - (g << 6)
                r3 = r * RPS
                for k in range(B // LANES):
                    x = d + k * LANES + iota
                    v = plsc.load_gather(sblk, [r3 + (x >> 6), x & 63])
                    ublk[r, pl.ds(k * LANES, LANES)] = v
                return carry

            lax.fori_loop(0, B, realign, 0, unroll=2)
            pltpu.make_async_copy(
                ublk, out_hbm.at[pl.ds(r0, B), pl.ds(c0, B)], SO[par]
            ).start()

            for k in range(B // LANES):
                iv = r0 + k * LANES + iota
                sv = iv * (N - 1) - ((iv * (iv - 1)) >> 1) + c0 - iv - 1
                dbuf[pl.ds(k * LANES, LANES)] = sv - (gbase_of(sv) << 6)

            def col_body(c, carry):
                for k in range(B // LANES):
                    rows3 = (k * LANES + iota) * RPS
                    dv = dbuf[pl.ds(k * LANES, LANES)]
                    x = dv + c
                    v = plsc.load_gather(sblk, [rows3 + (x >> 6), x & 63])
                    tblk[c, pl.ds(k * LANES, LANES)] = -v
                return carry

            lax.fori_loop(0, B, col_body, 0, unroll=2)
            pltpu.make_async_copy(
                tblk, out_hbm.at[pl.ds(c0, B), pl.ds(r0, B)], SO[par]
            ).start()

    fire_pair(0, 0)

    def loop_body(tt, carry):
        jt_a = 2 * tt
        fire_pair(jt_a + 1, 1)
        compute_pair(jt_a, tt, 0)

        nxt = jt_a + 2

        @pl.when(nxt == TSLOTS)
        def _():
            fire_block(diag_start, 0)

        @pl.when(nxt < TSLOTS)
        def _():
            fire_pair(nxt, 0)

        compute_pair(jt_a + 1, tt, 1)
        return carry

    lax.fori_loop(0, TSLOTS // 2, loop_body, 0)

    r0 = wid * B
    drain_in(0)
    drain_out(0)
    drain_out(0)
    sblk, tblk = SB[0], TB[0]

    for k in range(B // LANES):
        jv = k * LANES + iota
        gi = r0 + jv
        corr_j = jnp.where(gi == 0, 1, 0).astype(jnp.int32)
        sj = jnp.maximum(0, gi * (N - 1) - ((gi * (gi - 1)) >> 1) - jv - 1)
        dbuf[pl.ds(k * LANES, LANES)] = sj - (gbase_of(sj) << 6) - corr_j

    def drow(r, carry):
        i = r0 + r
        corr_r = jnp.where(i == 0, 1, 0).astype(jnp.int32)
        s_r = diag_start(r)
        d_r = s_r - (gbase_of(s_r) << 6)
        r3 = r * RPS
        for k in range(B // LANES):
            jv = k * LANES + iota
            xu = jnp.maximum(jv - corr_r, 0) + d_r
            vu = plsc.load_gather(sblk, [r3 + (xu >> 6), xu & 63])
            lbv = dbuf[pl.ds(k * LANES, LANES)]
            xl = jnp.maximum(r + lbv, 0)
            vl = plsc.load_gather(sblk, [jv * RPS + (xl >> 6), xl & 63])
            zero = jnp.zeros((LANES,), jnp.float32)
            val = jnp.where(jv > r, vu, zero) + jnp.where(jv < r, -vl, zero)
            tblk[r, pl.ds(k * LANES, LANES)] = val
        return carry

    lax.fori_loop(0, B, drow, 0, unroll=2)
    pltpu.make_async_copy(
        tblk, out_hbm.at[pl.ds(r0, B), pl.ds(r0, B)], SO[0]
    ).start()

    drain_out(1)
    drain_out(1)
    drain_out(0)


@jax.jit
def kernel(skewsym_params):
    mesh = plsc.VectorSubcoreMesh(core_axis_name="c", subcore_axis_name="s")
    f = pl.kernel(
        _body,
        out_type=jax.ShapeDtypeStruct((N, N), jnp.float32),
        mesh=mesh,
        compiler_params=pltpu.CompilerParams(
            use_tc_tiling_on_sc=False, needs_layout_passes=False
        ),
        scratch_types=[
            pltpu.VMEM((PADP,), jnp.int32),
            pltpu.VMEM((PADP,), jnp.int32),
            pltpu.VMEM((SROWS, D64), jnp.float32),
            pltpu.VMEM((SROWS, D64), jnp.float32),
            pltpu.VMEM((SROWS,), jnp.int32),
            pltpu.VMEM((SROWS,), jnp.int32),
            pltpu.VMEM((B, B), jnp.float32),
            pltpu.VMEM((B, B), jnp.float32),
            pltpu.VMEM((B, B), jnp.float32),
            pltpu.VMEM((B, B), jnp.float32),
            pltpu.VMEM((B,), jnp.int32),
            pltpu.SemaphoreType.DMA,
            pltpu.SemaphoreType.DMA,
            pltpu.SemaphoreType.DMA,
            pltpu.SemaphoreType.DMA,
        ],
    )
    return f(
        skewsym_params.reshape(M, D64),
        jnp.asarray(_bi_tab),
        jnp.asarray(_bj_tab),
    )


# speedup vs baseline: 1.8224x; 1.8224x over previous
import functools

import jax
import jax.numpy as jnp
import numpy as np
from jax import lax
from jax.experimental import pallas as pl
from jax.experimental.pallas import tpu as pltpu
from jax.experimental.pallas import tpu_sc as plsc

N = 4096
NPARAMS = N * (N - 1) // 2
B = 128
SW = B + 8
NB = N // B
NW = 32
LANES = 16

_pairs = [(bi, bj) for bi in range(NB) for bj in range(bi + 1, NB)]
NPAIR = len(_pairs)
TSLOTS = -(-NPAIR // NW)
PADP = NW * TSLOTS + LANES
_bi_tab = np.full((PADP,), 0, np.int32)
_bj_tab = np.full((PADP,), 1, np.int32)
for _p, (_a, _b) in enumerate(_pairs):
    _bi_tab[_p] = _a
    _bj_tab[_p] = _b


def _offset(i):
    return i * (N - 1) - ((i * (i - 1)) >> 1)


def _body(params_hbm, bi_hbm, bj_hbm, out_hbm, tabi, tabj,
          sblk0, sblk1, ublk0, ublk1, tblk0, tblk1,
          sem_in0, sem_in1, sem_out0, sem_out1):
    cid = lax.axis_index("c")
    sid = lax.axis_index("s")
    wid = sid * 2 + cid

    SB = (sblk0, sblk1)
    UB = (ublk0, ublk1)
    TB = (tblk0, tblk1)
    SI = (sem_in0, sem_in1)
    SO = (sem_out0, sem_out1)

    pltpu.sync_copy(bi_hbm, tabi)
    pltpu.sync_copy(bj_hbm, tabj)

    iota = lax.iota(jnp.int32, LANES)

    def shift_of(s):
        return jnp.maximum(s & 7, s - (NPARAMS - SW))

    def pair_of(p):
        bi = tabi[pl.ds(p, LANES)][0]
        bj = tabj[pl.ds(p, LANES)][0]
        return bi, bj

    def pair_start(bi, bj):
        def start_of_row(r):
            i = bi * B + r
            return _offset(i) + bj * B - i - 1

        return start_of_row

    def diag_start(r):
        return jnp.maximum(0, _offset(wid * B + r) - r - 1)

    def fire_rows(start_of_row, par):
        def fire(r, carry):
            s = start_of_row(r)
            sa = pl.multiple_of(jnp.minimum(s - (s & 7), NPARAMS - SW), 8)
            pltpu.make_async_copy(
                params_hbm.at[pl.ds(sa, SW)], SB[par].at[r], SI[par]
            ).start()
            return carry

        lax.fori_loop(0, B, fire, 0, unroll=4)

    def drain_in(par):
        pltpu.make_async_copy(
            out_hbm.at[pl.ds(0, B), pl.ds(0, SW)], SB[par], SI[par]
        ).wait()

    def drain_out(par):
        pltpu.make_async_copy(
            out_hbm.at[pl.ds(0, B), pl.ds(0, B)], UB[par], SO[par]
        ).wait()

    def fire_pair(jt, par):
        p = wid + NW * jt

        @pl.when(p < NPAIR)
        def _():
            bi, bj = pair_of(p)
            fire_rows(pair_start(bi, bj), par)

    def compute_pair(jt, tt, par):
        p = wid + NW * jt

        @pl.when(p < NPAIR)
        def _():
            drain_in(par)

            @pl.when(tt >= 1)
            def _():
                drain_out(par)
                drain_out(par)

            bi, bj = pair_of(p)
            r0 = pl.multiple_of(bi * B, B)
            c0 = pl.multiple_of(bj * B, B)
            start_of_row = pair_start(bi, bj)
            sblk, ublk, tblk = SB[par], UB[par], TB[par]

            def rbody(r, carry):
                d = shift_of(start_of_row(r))
                rsplat = jnp.full((LANES,), 0, jnp.int32) + r
                for k in range(B // LANES):
                    v = sblk[r, pl.ds(d + k * LANES, LANES)]
                    ublk[r, pl.ds(k * LANES, LANES)] = v
                    plsc.store_scatter(
                        tblk, [k * LANES + iota, rsplat], -v
                    )
                return carry

            lax.fori_loop(0, B, rbody, 0, unroll=2)
            pltpu.make_async_copy(
                ublk, out_hbm.at[pl.ds(r0, B), pl.ds(c0, B)], SO[par]
            ).start()
            pltpu.make_async_copy(
                tblk.at[:, pl.ds(0, B)],
                out_hbm.at[pl.ds(c0, B), pl.ds(r0, B)],
                SO[par],
            ).start()

    fire_pair(0, 0)

    def loop_body(tt, carry):
        jt_a = 2 * tt
        fire_pair(jt_a + 1, 1)
        compute_pair(jt_a, tt, 0)

        nxt = jt_a + 2

        @pl.when(nxt == TSLOTS)
        def _():
            fire_rows(diag_start, 0)

        @pl.when(nxt < TSLOTS)
        def _():
            fire_pair(nxt, 0)

        compute_pair(jt_a + 1, tt, 1)
        return carry

    lax.fori_loop(0, TSLOTS // 2, loop_body, 0)

    r0 = wid * B
    drain_in(0)
    drain_out(0)
    drain_out(0)
    sblk, tblk = SB[0], TB[0]

    def drow(rr, carry):
        r = (B - 1) - rr
        d = shift_of(diag_start(r))
        rsplat = jnp.full((LANES,), 0, jnp.int32) + r
        zero = jnp.zeros((LANES,), jnp.float32)
        for k in range(B // LANES):
            jv = k * LANES + iota
            v = sblk[r, pl.ds(d + k * LANES, LANES)]
            mask = jv > r
            tblk[r, pl.ds(k * LANES, LANES)] = jnp.where(mask, v, zero)
            plsc.store_scatter(tblk, [jv, rsplat], -v, mask=mask)
        return carry

    lax.fori_loop(0, B, drow, 0, unroll=2)

    @pl.when(wid == 0)
    def _():
        zsplat = jnp.full((LANES,), 0, jnp.int32)
        for k in range(B // LANES):
            jv = k * LANES + iota
            idx = jnp.maximum(jv - 1, 0)
            v = plsc.load_gather(sblk, [zsplat, idx])
            tblk[0, pl.ds(k * LANES, LANES)] = jnp.where(
                jv > 0, v, jnp.zeros((LANES,), jnp.float32)
            )
            plsc.store_scatter(tblk, [jv, zsplat], -v, mask=jv > 0)

    pltpu.make_async_copy(
        tblk.at[:, pl.ds(0, B)],
        out_hbm.at[pl.ds(r0, B), pl.ds(r0, B)],
        SO[0],
    ).start()

    drain_out(1)
    drain_out(1)
    drain_out(0)


@jax.jit
def kernel(skewsym_params):
    mesh = plsc.VectorSubcoreMesh(core_axis_name="c", subcore_axis_name="s")
    f = pl.kernel(
        _body,
        out_type=jax.ShapeDtypeStruct((N, N), jnp.float32),
        mesh=mesh,
        compiler_params=pltpu.CompilerParams(
            use_tc_tiling_on_sc=False, needs_layout_passes=False
        ),
        scratch_types=[
            pltpu.VMEM((PADP,), jnp.int32),
            pltpu.VMEM((PADP,), jnp.int32),
            pltpu.VMEM((B, SW), jnp.float32),
            pltpu.VMEM((B, SW), jnp.float32),
            pltpu.VMEM((B, B), jnp.float32),
            pltpu.VMEM((B, B), jnp.float32),
            pltpu.VMEM((B, SW), jnp.float32),
            pltpu.VMEM((B, SW), jnp.float32),
            pltpu.SemaphoreType.DMA,
            pltpu.SemaphoreType.DMA,
            pltpu.SemaphoreType.DMA,
            pltpu.SemaphoreType.DMA,
        ],
    )
    return f(skewsym_params, jnp.asarray(_bi_tab), jnp.asarray(_bj_tab))
